# Initial kernel scaffold; baseline (speedup 1.0000x reference)
import jax
import jax.numpy as jnp
from jax.experimental import pallas as pl


def kernel(table_fused, bias, sentence1, sentence2):
    raise NotImplementedError("write your pallas kernel here")


# speedup vs baseline: 2.4036x; 2.4036x over previous
import jax
import jax.numpy as jnp
from jax import lax
from jax.experimental import pallas as pl
from jax.experimental.pallas import tpu as pltpu


def _siamese_kernel(tok_ref, table_ref, bias_ref, out_ref, vec_ref):
    core = pl.program_id(0)
    n_vec = vec_ref.shape[0]
    pairs = out_ref.shape[0]
    seq = tok_ref.shape[1] // n_vec
    vpg = 8 if n_vec % 8 == 0 else 1

    def body(g, carry):
        for vloc in range(vpg):
            vec = g * vpg + vloc
            base = vec * seq
            m = table_ref[tok_ref[core, base], 0]
            for t in range(1, seq):
                m = jnp.maximum(m, table_ref[tok_ref[core, base + t], 0])
            vec_ref[vec, 0] = m
        return carry

    lax.fori_loop(0, n_vec // vpg, body, 0)

    pooled = vec_ref[...] + bias_ref[0, 0]
    v1 = pooled[:pairs]
    v2 = pooled[pairs:]
    eps2 = 1e-12 * 1e-12
    n1 = jnp.maximum(jnp.sum(v1 * v1, axis=2), eps2)
    n2 = jnp.maximum(jnp.sum(v2 * v2, axis=2), eps2)
    dt = jnp.sum(v1 * v2, axis=2)
    out_ref[...] = 1.0 - dt * lax.rsqrt(n1 * n2)


def kernel(table_fused, bias, sentence1, sentence2):
    v, h = table_fused.shape
    b, l = sentence1.shape
    n_cores = 2 if b % 2 == 0 else 1
    pairs = b // n_cores
    s1 = sentence1.astype(jnp.int32).reshape(n_cores, pairs, l)
    s2 = sentence2.astype(jnp.int32).reshape(n_cores, pairs, l)
    tok = jnp.concatenate([s1, s2], axis=1).reshape(n_cores, 2 * pairs * l)
    out = pl.pallas_call(
        _siamese_kernel,
        grid=(n_cores,),
        in_specs=[
            pl.BlockSpec(memory_space=pltpu.SMEM),
            pl.BlockSpec((v, 1, h), lambda i: (0, 0, 0)),
            pl.BlockSpec((1, 1, h), lambda i: (0, 0, 0)),
        ],
        out_specs=pl.BlockSpec((pairs, 1), lambda i: (i, 0)),
        out_shape=jax.ShapeDtypeStruct((b, 1), jnp.float32),
        scratch_shapes=[pltpu.VMEM((2 * pairs, 1, h), jnp.float32)],
        compiler_params=pltpu.CompilerParams(
            dimension_semantics=("parallel",),
            vmem_limit_bytes=48 * 1024 * 1024),
    )(tok, table_fused.reshape(v, 1, h), bias.reshape(1, 1, h))
    return out.reshape(-1)
